# baseline (speedup 1.0000x reference)
import functools

import jax
import jax.numpy as jnp
from jax import lax
from jax.experimental import pallas as pl
from jax.experimental.pallas import tpu as pltpu
from jax.experimental.pallas import tpu_sc as plsc

N = 10000
D = 128
H = 128
O = 64
G = 64

N_PAD = 10240
ROWS_PER_TILE = N_PAD // 16
CHUNK = 128
CPT = 80
E_PAD = 32 * CPT * CHUNK
NBUF = 2
SEG = 40
NSEG = CPT // SEG
BM = 1024
GRID = N_PAD // BM

_HIGH = jax.lax.Precision.HIGHEST


def _lrelu(x):
    return jnp.where(x >= 0, x, 0.01 * x)


DEGW = 128


DEG_K = 8


def _deg_half(cols_hbm, ones_hbm, zeros_hbm, deg_hbm, deg_sh, idx_v, ones_v,
              dsem, s, c):
    sl = pl.ds(s * ROWS_PER_TILE, ROWS_PER_TILE)
    pltpu.sync_copy(zeros_hbm, deg_sh.at[sl])
    plsc.subcore_barrier()
    g = s * 2 + c
    pltpu.sync_copy(cols_hbm.at[g], idx_v)

    def group_body(t, _):
        for k in range(DEG_K):
            j = t * DEG_K + k
            pltpu.async_copy(ones_v, deg_sh.at[idx_v.at[j]], dsem, add=True)
        for k in range(DEG_K):
            @pl.when(t > 0)
            def _():
                pltpu.make_async_copy(ones_hbm, ones_v, dsem).wait()
        return 0

    lax.fori_loop(0, CPT // DEG_K, group_body, 0)
    for k in range(DEG_K):
        pltpu.make_async_copy(ones_hbm, ones_v, dsem).wait()
    plsc.subcore_barrier()
    pltpu.sync_copy(deg_sh.at[sl], deg_hbm.at[sl])


@functools.partial(
    pl.kernel,
    out_type=(
        jax.ShapeDtypeStruct((N_PAD, DEGW), jnp.float32),
        jax.ShapeDtypeStruct((N_PAD, DEGW), jnp.float32),
    ),
    mesh=plsc.VectorSubcoreMesh(core_axis_name="c", subcore_axis_name="s"),
    scratch_types=[
        pltpu.VMEM_SHARED((N_PAD, DEGW), jnp.float32),
        pltpu.VMEM((CPT, CHUNK), jnp.int32),
        pltpu.VMEM((CHUNK, DEGW), jnp.float32),
        pltpu.SemaphoreType.DMA,
    ],
)
def _deg_kernel(cols_hbm, ones_hbm, zeros_hbm, deg0, deg1, deg_sh, idx_v,
                ones_v, dsem):
    c = lax.axis_index("c")
    s = lax.axis_index("s")
    pltpu.sync_copy(ones_hbm, ones_v)

    @pl.when(c == 0)
    def _():
        _deg_half(cols_hbm, ones_hbm, zeros_hbm, deg0, deg_sh, idx_v, ones_v,
                  dsem, s, 0)

    @pl.when(c == 1)
    def _():
        _deg_half(cols_hbm, ones_hbm, zeros_hbm, deg1, deg_sh, idx_v, ones_v,
                  dsem, s, 1)


def _msg_half(xp_hbm, zeros_hbm, y_hbm, rows_hbm, cols_hbm,
              y_sh, idxr_v, idxc_v, buf, sem, s, c):
    sl = pl.ds(s * ROWS_PER_TILE, ROWS_PER_TILE)
    pltpu.sync_copy(zeros_hbm, y_sh.at[sl])
    plsc.subcore_barrier()
    g = s * 2 + c
    pltpu.sync_copy(rows_hbm.at[g], idxr_v)
    pltpu.sync_copy(cols_hbm.at[g], idxc_v)

    def chunk_body(j, _):
        pltpu.async_copy(xp_hbm.at[idxr_v.at[j]], buf, sem).wait()
        pltpu.sync_copy(buf, y_sh.at[idxc_v.at[j]], add=True)
        return 0

    lax.fori_loop(0, CPT, chunk_body, 0)
    plsc.subcore_barrier()
    pltpu.sync_copy(y_sh.at[sl], y_hbm.at[sl])


@functools.partial(
    pl.kernel,
    out_type=(
        jax.ShapeDtypeStruct((N_PAD, H), jnp.float32),
        jax.ShapeDtypeStruct((N_PAD, H), jnp.float32),
    ),
    mesh=plsc.VectorSubcoreMesh(core_axis_name="c", subcore_axis_name="s"),
    scratch_types=[
        pltpu.VMEM_SHARED((N_PAD, H), jnp.float32),
        pltpu.VMEM((CPT, CHUNK), jnp.int32),
        pltpu.VMEM((CPT, CHUNK), jnp.int32),
        pltpu.VMEM((CHUNK, H), jnp.float32),
        pltpu.SemaphoreType.DMA,
    ],
)
def _msg_kernel(xp, zeros_hbm, rows_hbm, cols_hbm, y0, y1,
                y_sh, idxr_v, idxc_v, buf, sem):
    c = lax.axis_index("c")
    s = lax.axis_index("s")

    @pl.when(c == 0)
    def _():
        _msg_half(xp, zeros_hbm, y0, rows_hbm, cols_hbm,
                  y_sh, idxr_v, idxc_v, buf, sem, s, 0)

    @pl.when(c == 1)
    def _():
        _msg_half(xp, zeros_hbm, y1, rows_hbm, cols_hbm,
                  y_sh, idxr_v, idxc_v, buf, sem, s, 1)


def _dis_from_parts(d0_ref, d1_ref):
    deg = d0_ref[...][:, 0] + d1_ref[...][:, 0] + 1.0
    return lax.rsqrt(deg)


def _tc1_body(feat_ref, d0_ref, d1_ref, wemb_ref, wm1_ref, xp_ref):
    dis = _dis_from_parts(d0_ref, d1_ref)
    h0 = _lrelu(jnp.dot(feat_ref[...], wemb_ref[...],
                        preferred_element_type=jnp.float32, precision=_HIGH))
    x = jnp.dot(h0, wm1_ref[...],
                preferred_element_type=jnp.float32, precision=_HIGH)
    xp_ref[...] = x * dis[:, None]


def _tc1_call(features, deg0, deg1, W_emb, W_m1):
    return pl.pallas_call(
        _tc1_body,
        grid=(GRID,),
        in_specs=[
            pl.BlockSpec((BM, D), lambda i: (i, 0)),
            pl.BlockSpec((BM, DEGW), lambda i: (i, 0)),
            pl.BlockSpec((BM, DEGW), lambda i: (i, 0)),
            pl.BlockSpec((D, H), lambda i: (0, 0)),
            pl.BlockSpec((H, H), lambda i: (0, 0)),
        ],
        out_specs=pl.BlockSpec((BM, H), lambda i: (i, 0)),
        out_shape=jax.ShapeDtypeStruct((N_PAD, H), jnp.float32),
    )(features, deg0, deg1, W_emb, W_m1)


def _tc2_body(y0_ref, y1_ref, xpin_ref, d0_ref, d1_ref, w_ref, xp_ref):
    dis = _dis_from_parts(d0_ref, d1_ref)
    y = y0_ref[...] + y1_ref[...] + xpin_ref[...]
    h = _lrelu(y * dis[:, None])
    x = jnp.dot(h, w_ref[...],
                preferred_element_type=jnp.float32, precision=_HIGH)
    xp_ref[...] = x * dis[:, None]


def _tc2_call(y0, y1, xpin, deg0, deg1, W):
    return pl.pallas_call(
        _tc2_body,
        grid=(GRID,),
        in_specs=[
            pl.BlockSpec((BM, H), lambda i: (i, 0)),
            pl.BlockSpec((BM, H), lambda i: (i, 0)),
            pl.BlockSpec((BM, H), lambda i: (i, 0)),
            pl.BlockSpec((BM, DEGW), lambda i: (i, 0)),
            pl.BlockSpec((BM, DEGW), lambda i: (i, 0)),
            pl.BlockSpec((H, H), lambda i: (0, 0)),
        ],
        out_specs=pl.BlockSpec((BM, H), lambda i: (i, 0)),
        out_shape=jax.ShapeDtypeStruct((N_PAD, H), jnp.float32),
    )(y0, y1, xpin, deg0, deg1, W)


def _tc3_body(y0_ref, y1_ref, xpin_ref, d0_ref, d1_ref, batch_ref,
              wr1_ref, wr2_ref, out_ref, sums_acc, cnt_acc):
    i = pl.program_id(0)

    @pl.when(i == 0)
    def _():
        sums_acc[...] = jnp.zeros_like(sums_acc)
        cnt_acc[...] = jnp.zeros_like(cnt_acc)

    dis = _dis_from_parts(d0_ref, d1_ref)
    y = y0_ref[...] + y1_ref[...] + xpin_ref[...]
    h = _lrelu(y * dis[:, None])
    rowid = lax.broadcasted_iota(jnp.int32, (BM, 1), 0) + i * BM
    h = jnp.where(rowid < N, h, 0.0)
    b = batch_ref[0, 0, :]
    onehot = (b[:, None] == lax.broadcasted_iota(jnp.int32, (1, G), 1)
              ).astype(jnp.float32)
    sums_acc[...] += lax.dot_general(
        onehot, h, (((0,), (0,)), ((), ())),
        preferred_element_type=jnp.float32, precision=_HIGH)
    cnt_acc[...] += lax.dot_general(
        onehot, jnp.ones_like(h), (((0,), (0,)), ((), ())),
        preferred_element_type=jnp.float32, precision=_HIGH)

    @pl.when(i == GRID - 1)
    def _():
        mean = sums_acc[...] / jnp.maximum(cnt_acc[...], 1.0)
        z = _lrelu(jnp.dot(mean, wr1_ref[...],
                           preferred_element_type=jnp.float32, precision=_HIGH))
        z = jnp.dot(z, wr2_ref[...],
                    preferred_element_type=jnp.float32, precision=_HIGH)
        out_ref[...] = 1.0 / (1.0 + jnp.exp(-z))


def _tc3_call(y0, y1, xpin, deg0, deg1, batch3d, W_r1, W_r2):
    return pl.pallas_call(
        _tc3_body,
        grid=(GRID,),
        in_specs=[
            pl.BlockSpec((BM, H), lambda i: (i, 0)),
            pl.BlockSpec((BM, H), lambda i: (i, 0)),
            pl.BlockSpec((BM, H), lambda i: (i, 0)),
            pl.BlockSpec((BM, DEGW), lambda i: (i, 0)),
            pl.BlockSpec((BM, DEGW), lambda i: (i, 0)),
            pl.BlockSpec((1, 1, BM), lambda i: (i, 0, 0)),
            pl.BlockSpec((H, H), lambda i: (0, 0)),
            pl.BlockSpec((H, O), lambda i: (0, 0)),
        ],
        out_specs=pl.BlockSpec((G, O), lambda i: (0, 0)),
        out_shape=jax.ShapeDtypeStruct((G, O), jnp.float32),
        scratch_shapes=[
            pltpu.VMEM((G, H), jnp.float32),
            pltpu.VMEM((G, H), jnp.float32),
        ],
        compiler_params=pltpu.CompilerParams(
            dimension_semantics=("arbitrary",)),
    )(y0, y1, xpin, deg0, deg1, batch3d, W_r1, W_r2)


def kernel(features, edge_index, batch, W_emb, W_m1, W_m2, W_r1, W_r2):
    row = edge_index[0]
    col = edge_index[1]
    n_extra = E_PAD - row.shape[0]
    row_p = jnp.concatenate([row, jnp.zeros((n_extra,), jnp.int32)])
    col_p = jnp.concatenate([col, jnp.full((n_extra,), N, jnp.int32)])
    rows_l = row_p.reshape(32, CPT, CHUNK)
    cols_l = col_p.reshape(32, CPT, CHUNK)
    batch_p = jnp.concatenate(
        [batch, jnp.full((N_PAD - N,), G, jnp.int32)])
    batch3d = batch_p.reshape(GRID, 1, BM)

    ones_deg = jnp.ones((CHUNK, DEGW), jnp.float32)
    zeros_deg = jnp.zeros((ROWS_PER_TILE, DEGW), jnp.float32)
    zeros_msg = jnp.zeros((ROWS_PER_TILE, H), jnp.float32)
    deg0, deg1 = _deg_kernel(cols_l, ones_deg, zeros_deg)

    xp = _tc1_call(features, deg0, deg1, W_emb, W_m1)
    y0, y1 = _msg_kernel(xp, zeros_msg, rows_l, cols_l)
    xp2 = _tc2_call(y0, y1, xp, deg0, deg1, W_m2)
    y0, y1 = _msg_kernel(xp2, zeros_msg, rows_l, cols_l)
    return _tc3_call(y0, y1, xp2, deg0, deg1, batch3d, W_r1, W_r2)


# speedup vs baseline: 2.4038x; 2.4038x over previous
import functools

import jax
import jax.numpy as jnp
from jax import lax
from jax.experimental import pallas as pl
from jax.experimental.pallas import tpu as pltpu
from jax.experimental.pallas import tpu_sc as plsc

N = 10000
D = 128
H = 128
O = 64
G = 64

N_PAD = 10240
ROWS_PER_TILE = N_PAD // 16
CHUNK = 128
CPT = 80
E_PAD = 32 * CPT * CHUNK
NBUF = 2
SEG = 40
NSEG = CPT // SEG
BM = 1024
GRID = N_PAD // BM

_HIGH = jax.lax.Precision.HIGHEST


def _lrelu(x):
    return jnp.where(x >= 0, x, 0.01 * x)


DEGW = 128


DEG_K = 8


def _deg_half(cols_hbm, ones_hbm, zeros_hbm, deg_hbm, deg_sh, idx_v, ones_v,
              dsem, s, c):
    sl = pl.ds(s * ROWS_PER_TILE, ROWS_PER_TILE)
    pltpu.sync_copy(zeros_hbm, deg_sh.at[sl])
    plsc.subcore_barrier()
    g = s * 2 + c
    pltpu.sync_copy(cols_hbm.at[g], idx_v)

    def group_body(t, _):
        for k in range(DEG_K):
            j = t * DEG_K + k
            pltpu.async_copy(ones_v, deg_sh.at[idx_v.at[j]], dsem, add=True)
        for k in range(DEG_K):
            @pl.when(t > 0)
            def _():
                pltpu.make_async_copy(ones_hbm, ones_v, dsem).wait()
        return 0

    lax.fori_loop(0, CPT // DEG_K, group_body, 0)
    for k in range(DEG_K):
        pltpu.make_async_copy(ones_hbm, ones_v, dsem).wait()
    plsc.subcore_barrier()
    pltpu.sync_copy(deg_sh.at[sl], deg_hbm.at[sl])


@functools.partial(
    pl.kernel,
    out_type=(
        jax.ShapeDtypeStruct((N_PAD, DEGW), jnp.float32),
        jax.ShapeDtypeStruct((N_PAD, DEGW), jnp.float32),
    ),
    mesh=plsc.VectorSubcoreMesh(core_axis_name="c", subcore_axis_name="s"),
    scratch_types=[
        pltpu.VMEM_SHARED((N_PAD, DEGW), jnp.float32),
        pltpu.VMEM((CPT, CHUNK), jnp.int32),
        pltpu.VMEM((CHUNK, DEGW), jnp.float32),
        pltpu.SemaphoreType.DMA,
    ],
)
def _deg_kernel(cols_hbm, ones_hbm, zeros_hbm, deg0, deg1, deg_sh, idx_v,
                ones_v, dsem):
    c = lax.axis_index("c")
    s = lax.axis_index("s")
    pltpu.sync_copy(ones_hbm, ones_v)

    @pl.when(c == 0)
    def _():
        _deg_half(cols_hbm, ones_hbm, zeros_hbm, deg0, deg_sh, idx_v, ones_v,
                  dsem, s, 0)

    @pl.when(c == 1)
    def _():
        _deg_half(cols_hbm, ones_hbm, zeros_hbm, deg1, deg_sh, idx_v, ones_v,
                  dsem, s, 1)


def _msg_half(xp_hbm, y_hbm, rows_hbm, cols_hbm,
              y_sh, idxr_v, idxc_v, buf, sem, s, c):
    z16 = jnp.zeros((16,), jnp.float32)

    def zero_row(r, _):
        for k in range(8):
            buf[r, pl.ds(k * 16, 16)] = z16
        return 0

    lax.fori_loop(0, CHUNK, zero_row, 0)
    for q in range(ROWS_PER_TILE // CHUNK):
        pltpu.sync_copy(buf, y_sh.at[pl.ds(s * ROWS_PER_TILE + q * CHUNK,
                                           CHUNK)])
    sl = pl.ds(s * ROWS_PER_TILE, ROWS_PER_TILE)
    plsc.subcore_barrier()
    g = s * 2 + c
    pltpu.sync_copy(rows_hbm.at[g], idxr_v)
    pltpu.sync_copy(cols_hbm.at[g], idxc_v)

    def chunk_body(j, _):
        pltpu.async_copy(xp_hbm.at[idxr_v.at[j]], buf, sem).wait()
        pltpu.sync_copy(buf, y_sh.at[idxc_v.at[j]], add=True)
        return 0

    lax.fori_loop(0, CPT, chunk_body, 0)
    plsc.subcore_barrier()
    pltpu.sync_copy(y_sh.at[sl], y_hbm.at[sl])


@functools.partial(
    pl.kernel,
    out_type=(
        jax.ShapeDtypeStruct((N_PAD, H), jnp.float32),
        jax.ShapeDtypeStruct((N_PAD, H), jnp.float32),
    ),
    mesh=plsc.VectorSubcoreMesh(core_axis_name="c", subcore_axis_name="s"),
    scratch_types=[
        pltpu.VMEM_SHARED((N_PAD, H), jnp.float32),
        pltpu.VMEM((CPT, CHUNK), jnp.int32),
        pltpu.VMEM((CPT, CHUNK), jnp.int32),
        pltpu.VMEM((CHUNK, H), jnp.float32),
        pltpu.SemaphoreType.DMA,
    ],
)
def _msg_kernel(xp, rows_hbm, cols_hbm, y0, y1,
                y_sh, idxr_v, idxc_v, buf, sem):
    c = lax.axis_index("c")
    s = lax.axis_index("s")

    @pl.when(c == 0)
    def _():
        _msg_half(xp, y0, rows_hbm, cols_hbm,
                  y_sh, idxr_v, idxc_v, buf, sem, s, 0)

    @pl.when(c == 1)
    def _():
        _msg_half(xp, y1, rows_hbm, cols_hbm,
                  y_sh, idxr_v, idxc_v, buf, sem, s, 1)


def _dis_from_parts(d0_ref, d1_ref):
    deg = d0_ref[...][:, 0] + d1_ref[...][:, 0] + 1.0
    return lax.rsqrt(deg)


def _tc1_body(feat_ref, d0_ref, d1_ref, wemb_ref, wm1_ref, xp_ref):
    dis = _dis_from_parts(d0_ref, d1_ref)
    h0 = _lrelu(jnp.dot(feat_ref[...], wemb_ref[...],
                        preferred_element_type=jnp.float32, precision=_HIGH))
    x = jnp.dot(h0, wm1_ref[...],
                preferred_element_type=jnp.float32, precision=_HIGH)
    xp_ref[...] = x * dis[:, None]


def _tc1_call(features, deg0, deg1, W_emb, W_m1):
    return pl.pallas_call(
        _tc1_body,
        grid=(GRID,),
        in_specs=[
            pl.BlockSpec((BM, D), lambda i: (i, 0)),
            pl.BlockSpec((BM, DEGW), lambda i: (i, 0)),
            pl.BlockSpec((BM, DEGW), lambda i: (i, 0)),
            pl.BlockSpec((D, H), lambda i: (0, 0)),
            pl.BlockSpec((H, H), lambda i: (0, 0)),
        ],
        out_specs=pl.BlockSpec((BM, H), lambda i: (i, 0)),
        out_shape=jax.ShapeDtypeStruct((N_PAD, H), jnp.float32),
    )(features, deg0, deg1, W_emb, W_m1)


def _tc2_body(y0_ref, y1_ref, xpin_ref, d0_ref, d1_ref, w_ref, xp_ref):
    dis = _dis_from_parts(d0_ref, d1_ref)
    y = y0_ref[...] + y1_ref[...] + xpin_ref[...]
    h = _lrelu(y * dis[:, None])
    x = jnp.dot(h, w_ref[...],
                preferred_element_type=jnp.float32, precision=_HIGH)
    xp_ref[...] = x * dis[:, None]


def _tc2_call(y0, y1, xpin, deg0, deg1, W):
    return pl.pallas_call(
        _tc2_body,
        grid=(GRID,),
        in_specs=[
            pl.BlockSpec((BM, H), lambda i: (i, 0)),
            pl.BlockSpec((BM, H), lambda i: (i, 0)),
            pl.BlockSpec((BM, H), lambda i: (i, 0)),
            pl.BlockSpec((BM, DEGW), lambda i: (i, 0)),
            pl.BlockSpec((BM, DEGW), lambda i: (i, 0)),
            pl.BlockSpec((H, H), lambda i: (0, 0)),
        ],
        out_specs=pl.BlockSpec((BM, H), lambda i: (i, 0)),
        out_shape=jax.ShapeDtypeStruct((N_PAD, H), jnp.float32),
    )(y0, y1, xpin, deg0, deg1, W)


def _tc3_body(y0_ref, y1_ref, xpin_ref, d0_ref, d1_ref, batch_ref,
              wr1_ref, wr2_ref, out_ref, sums_acc, cnt_acc):
    i = pl.program_id(0)

    @pl.when(i == 0)
    def _():
        sums_acc[...] = jnp.zeros_like(sums_acc)
        cnt_acc[...] = jnp.zeros_like(cnt_acc)

    dis = _dis_from_parts(d0_ref, d1_ref)
    y = y0_ref[...] + y1_ref[...] + xpin_ref[...]
    h = _lrelu(y * dis[:, None])
    rowid = lax.broadcasted_iota(jnp.int32, (BM, 1), 0) + i * BM
    h = jnp.where(rowid < N, h, 0.0)
    b = batch_ref[0, 0, :]
    onehot = (b[:, None] == lax.broadcasted_iota(jnp.int32, (1, G), 1)
              ).astype(jnp.float32)
    sums_acc[...] += lax.dot_general(
        onehot, h, (((0,), (0,)), ((), ())),
        preferred_element_type=jnp.float32, precision=_HIGH)
    cnt_acc[...] += lax.dot_general(
        onehot, jnp.ones_like(h), (((0,), (0,)), ((), ())),
        preferred_element_type=jnp.float32, precision=_HIGH)

    @pl.when(i == GRID - 1)
    def _():
        mean = sums_acc[...] / jnp.maximum(cnt_acc[...], 1.0)
        z = _lrelu(jnp.dot(mean, wr1_ref[...],
                           preferred_element_type=jnp.float32, precision=_HIGH))
        z = jnp.dot(z, wr2_ref[...],
                    preferred_element_type=jnp.float32, precision=_HIGH)
        out_ref[...] = 1.0 / (1.0 + jnp.exp(-z))


def _tc3_call(y0, y1, xpin, deg0, deg1, batch3d, W_r1, W_r2):
    return pl.pallas_call(
        _tc3_body,
        grid=(GRID,),
        in_specs=[
            pl.BlockSpec((BM, H), lambda i: (i, 0)),
            pl.BlockSpec((BM, H), lambda i: (i, 0)),
            pl.BlockSpec((BM, H), lambda i: (i, 0)),
            pl.BlockSpec((BM, DEGW), lambda i: (i, 0)),
            pl.BlockSpec((BM, DEGW), lambda i: (i, 0)),
            pl.BlockSpec((1, 1, BM), lambda i: (i, 0, 0)),
            pl.BlockSpec((H, H), lambda i: (0, 0)),
            pl.BlockSpec((H, O), lambda i: (0, 0)),
        ],
        out_specs=pl.BlockSpec((G, O), lambda i: (0, 0)),
        out_shape=jax.ShapeDtypeStruct((G, O), jnp.float32),
        scratch_shapes=[
            pltpu.VMEM((G, H), jnp.float32),
            pltpu.VMEM((G, H), jnp.float32),
        ],
        compiler_params=pltpu.CompilerParams(
            dimension_semantics=("arbitrary",)),
    )(y0, y1, xpin, deg0, deg1, batch3d, W_r1, W_r2)


def kernel(features, edge_index, batch, W_emb, W_m1, W_m2, W_r1, W_r2):
    row = edge_index[0]
    col = edge_index[1]
    n_extra = E_PAD - row.shape[0]
    pad_idx = jnp.arange(n_extra, dtype=jnp.int32)
    row_p = jnp.concatenate([row, pad_idx % N])
    col_p = jnp.concatenate([col, N + pad_idx % (N_PAD - N)])
    rows_l = row_p.reshape(32, CPT, CHUNK)
    cols_l = col_p.reshape(32, CPT, CHUNK)
    batch_p = jnp.concatenate(
        [batch, jnp.full((N_PAD - N,), G, jnp.int32)])
    batch3d = batch_p.reshape(GRID, 1, BM)

    ones_deg = jnp.ones((CHUNK, DEGW), jnp.float32)
    zeros_deg = jnp.zeros((ROWS_PER_TILE, DEGW), jnp.float32)
    deg0, deg1 = _deg_kernel(cols_l, ones_deg, zeros_deg)

    xp = _tc1_call(features, deg0, deg1, W_emb, W_m1)
    y0, y1 = _msg_kernel(xp, rows_l, cols_l)
    xp2 = _tc2_call(y0, y1, xp, deg0, deg1, W_m2)
    y0, y1 = _msg_kernel(xp2, rows_l, cols_l)
    return _tc3_call(y0, y1, xp2, deg0, deg1, batch3d, W_r1, W_r2)


# speedup vs baseline: 2.6406x; 1.0985x over previous
import functools

import jax
import jax.numpy as jnp
from jax import lax
from jax.experimental import pallas as pl
from jax.experimental.pallas import tpu as pltpu
from jax.experimental.pallas import tpu_sc as plsc

N = 10000
D = 128
H = 128
O = 64
G = 64

N_PAD = 10240
ROWS_PER_TILE = N_PAD // 16
CHUNK = 128
CPT = 80
E_PAD = 32 * CPT * CHUNK
NBUF = 2
SEG = 40
NSEG = CPT // SEG
BM = 1024
GRID = N_PAD // BM

_HIGH = jax.lax.Precision.HIGHEST


def _lrelu(x):
    return jnp.where(x >= 0, x, 0.01 * x)


DEGW = 128


DEG_K = 8


def _deg_half(cols_hbm, ones_hbm, deg_hbm, deg_sh, idx_v, ones_v,
              dsem, s, c):
    sl = pl.ds(s * ROWS_PER_TILE, ROWS_PER_TILE)
    z16 = jnp.zeros((16,), jnp.float32)

    def zero_row(r, _):
        for k in range(DEGW // 16):
            ones_v[r, pl.ds(k * 16, 16)] = z16
        return 0

    lax.fori_loop(0, CHUNK, zero_row, 0)
    for q in range(ROWS_PER_TILE // CHUNK):
        pltpu.sync_copy(ones_v, deg_sh.at[pl.ds(s * ROWS_PER_TILE + q * CHUNK,
                                                CHUNK)])
    pltpu.sync_copy(ones_hbm, ones_v)
    plsc.subcore_barrier()
    g = s * 2 + c
    pltpu.sync_copy(cols_hbm.at[g], idx_v)

    def group_body(t, _):
        for k in range(DEG_K):
            j = t * DEG_K + k
            pltpu.async_copy(ones_v, deg_sh.at[idx_v.at[j]], dsem, add=True)
        for k in range(DEG_K):
            @pl.when(t > 0)
            def _():
                pltpu.make_async_copy(ones_hbm, ones_v, dsem).wait()
        return 0

    lax.fori_loop(0, CPT // DEG_K, group_body, 0)
    for k in range(DEG_K):
        pltpu.make_async_copy(ones_hbm, ones_v, dsem).wait()
    plsc.subcore_barrier()
    pltpu.sync_copy(deg_sh.at[sl], deg_hbm.at[sl])


@functools.partial(
    pl.kernel,
    out_type=(
        jax.ShapeDtypeStruct((N_PAD, DEGW), jnp.float32),
        jax.ShapeDtypeStruct((N_PAD, DEGW), jnp.float32),
    ),
    mesh=plsc.VectorSubcoreMesh(core_axis_name="c", subcore_axis_name="s"),
    scratch_types=[
        pltpu.VMEM_SHARED((N_PAD, DEGW), jnp.float32),
        pltpu.VMEM((CPT, CHUNK), jnp.int32),
        pltpu.VMEM((CHUNK, DEGW), jnp.float32),
        pltpu.SemaphoreType.DMA,
    ],
)
def _deg_kernel(cols_hbm, ones_hbm, deg0, deg1, deg_sh, idx_v,
                ones_v, dsem):
    c = lax.axis_index("c")
    s = lax.axis_index("s")

    @pl.when(c == 0)
    def _():
        _deg_half(cols_hbm, ones_hbm, deg0, deg_sh, idx_v, ones_v,
                  dsem, s, 0)

    @pl.when(c == 1)
    def _():
        _deg_half(cols_hbm, ones_hbm, deg1, deg_sh, idx_v, ones_v,
                  dsem, s, 1)


MSG_SEG = 16


def _msg_half(xp_hbm, y_hbm, rows_hbm, cols_hbm,
              y_sh, idxr_v, idxc_v, buf0, buf1, g0, g1, s0, s1, s, c):
    z16 = jnp.zeros((16,), jnp.float32)

    def zero_row(r, _):
        for k in range(8):
            buf0[r, pl.ds(k * 16, 16)] = z16
        return 0

    lax.fori_loop(0, CHUNK, zero_row, 0)
    for q in range(ROWS_PER_TILE // CHUNK):
        pltpu.sync_copy(buf0, y_sh.at[pl.ds(s * ROWS_PER_TILE + q * CHUNK,
                                            CHUNK)])
    sl = pl.ds(s * ROWS_PER_TILE, ROWS_PER_TILE)
    plsc.subcore_barrier()
    g = s * 2 + c
    for seg in range(CPT // MSG_SEG):
        base = seg * MSG_SEG
        pltpu.sync_copy(rows_hbm.at[g, pl.ds(base, MSG_SEG)], idxr_v)
        pltpu.sync_copy(cols_hbm.at[g, pl.ds(base, MSG_SEG)], idxc_v)

        def round_body(t, _):
            j = t * 2
            pltpu.async_copy(xp_hbm.at[idxr_v.at[j]], buf0, g0)
            pltpu.async_copy(xp_hbm.at[idxr_v.at[j + 1]], buf1, g1)
            pltpu.make_async_copy(xp_hbm.at[pl.ds(0, CHUNK)], buf0, g0).wait()
            pltpu.make_async_copy(xp_hbm.at[pl.ds(0, CHUNK)], buf1, g1).wait()
            pltpu.async_copy(buf0, y_sh.at[idxc_v.at[j]], s0, add=True)
            pltpu.async_copy(buf1, y_sh.at[idxc_v.at[j + 1]], s1, add=True)
            pltpu.make_async_copy(xp_hbm.at[pl.ds(0, CHUNK)], buf0, s0).wait()
            pltpu.make_async_copy(xp_hbm.at[pl.ds(0, CHUNK)], buf1, s1).wait()
            return 0

        lax.fori_loop(0, MSG_SEG // 2, round_body, 0)
    plsc.subcore_barrier()
    pltpu.sync_copy(y_sh.at[sl], y_hbm.at[sl])


@functools.partial(
    pl.kernel,
    out_type=(
        jax.ShapeDtypeStruct((N_PAD, H), jnp.float32),
        jax.ShapeDtypeStruct((N_PAD, H), jnp.float32),
    ),
    mesh=plsc.VectorSubcoreMesh(core_axis_name="c", subcore_axis_name="s"),
    scratch_types=[
        pltpu.VMEM_SHARED((N_PAD, H), jnp.float32),
        pltpu.VMEM((MSG_SEG, CHUNK), jnp.int32),
        pltpu.VMEM((MSG_SEG, CHUNK), jnp.int32),
        pltpu.VMEM((CHUNK, H), jnp.float32),
        pltpu.VMEM((CHUNK, H), jnp.float32),
        pltpu.SemaphoreType.DMA,
        pltpu.SemaphoreType.DMA,
        pltpu.SemaphoreType.DMA,
        pltpu.SemaphoreType.DMA,
    ],
)
def _msg_kernel(xp, rows_hbm, cols_hbm, y0, y1,
                y_sh, idxr_v, idxc_v, buf0, buf1, g0, g1, s0, s1):
    c = lax.axis_index("c")
    s = lax.axis_index("s")

    @pl.when(c == 0)
    def _():
        _msg_half(xp, y0, rows_hbm, cols_hbm,
                  y_sh, idxr_v, idxc_v, buf0, buf1, g0, g1, s0, s1, s, 0)

    @pl.when(c == 1)
    def _():
        _msg_half(xp, y1, rows_hbm, cols_hbm,
                  y_sh, idxr_v, idxc_v, buf0, buf1, g0, g1, s0, s1, s, 1)


def _dis_from_parts(d0_ref, d1_ref):
    deg = d0_ref[...][:, 0] + d1_ref[...][:, 0] + 1.0
    return lax.rsqrt(deg)


def _tc1_body(feat_ref, d0_ref, d1_ref, wemb_ref, wm1_ref, xp_ref):
    dis = _dis_from_parts(d0_ref, d1_ref)
    h0 = _lrelu(jnp.dot(feat_ref[...], wemb_ref[...],
                        preferred_element_type=jnp.float32, precision=_HIGH))
    x = jnp.dot(h0, wm1_ref[...],
                preferred_element_type=jnp.float32, precision=_HIGH)
    xp_ref[...] = x * dis[:, None]


def _tc1_call(features, deg0, deg1, W_emb, W_m1):
    return pl.pallas_call(
        _tc1_body,
        grid=(GRID,),
        in_specs=[
            pl.BlockSpec((BM, D), lambda i: (i, 0)),
            pl.BlockSpec((BM, DEGW), lambda i: (i, 0)),
            pl.BlockSpec((BM, DEGW), lambda i: (i, 0)),
            pl.BlockSpec((D, H), lambda i: (0, 0)),
            pl.BlockSpec((H, H), lambda i: (0, 0)),
        ],
        out_specs=pl.BlockSpec((BM, H), lambda i: (i, 0)),
        out_shape=jax.ShapeDtypeStruct((N_PAD, H), jnp.float32),
    )(features, deg0, deg1, W_emb, W_m1)


def _tc2_body(y0_ref, y1_ref, xpin_ref, d0_ref, d1_ref, w_ref, xp_ref):
    dis = _dis_from_parts(d0_ref, d1_ref)
    y = y0_ref[...] + y1_ref[...] + xpin_ref[...]
    h = _lrelu(y * dis[:, None])
    x = jnp.dot(h, w_ref[...],
                preferred_element_type=jnp.float32, precision=_HIGH)
    xp_ref[...] = x * dis[:, None]


def _tc2_call(y0, y1, xpin, deg0, deg1, W):
    return pl.pallas_call(
        _tc2_body,
        grid=(GRID,),
        in_specs=[
            pl.BlockSpec((BM, H), lambda i: (i, 0)),
            pl.BlockSpec((BM, H), lambda i: (i, 0)),
            pl.BlockSpec((BM, H), lambda i: (i, 0)),
            pl.BlockSpec((BM, DEGW), lambda i: (i, 0)),
            pl.BlockSpec((BM, DEGW), lambda i: (i, 0)),
            pl.BlockSpec((H, H), lambda i: (0, 0)),
        ],
        out_specs=pl.BlockSpec((BM, H), lambda i: (i, 0)),
        out_shape=jax.ShapeDtypeStruct((N_PAD, H), jnp.float32),
    )(y0, y1, xpin, deg0, deg1, W)


def _tc3_body(y0_ref, y1_ref, xpin_ref, d0_ref, d1_ref, batch_ref,
              wr1_ref, wr2_ref, out_ref, sums_acc, cnt_acc):
    i = pl.program_id(0)

    @pl.when(i == 0)
    def _():
        sums_acc[...] = jnp.zeros_like(sums_acc)
        cnt_acc[...] = jnp.zeros_like(cnt_acc)

    dis = _dis_from_parts(d0_ref, d1_ref)
    y = y0_ref[...] + y1_ref[...] + xpin_ref[...]
    h = _lrelu(y * dis[:, None])
    rowid = lax.broadcasted_iota(jnp.int32, (BM, 1), 0) + i * BM
    h = jnp.where(rowid < N, h, 0.0)
    b = batch_ref[0, 0, :]
    onehot = (b[:, None] == lax.broadcasted_iota(jnp.int32, (1, G), 1)
              ).astype(jnp.float32)
    sums_acc[...] += lax.dot_general(
        onehot, h, (((0,), (0,)), ((), ())),
        preferred_element_type=jnp.float32, precision=_HIGH)
    cnt_acc[...] += lax.dot_general(
        onehot, jnp.ones_like(h), (((0,), (0,)), ((), ())),
        preferred_element_type=jnp.float32, precision=_HIGH)

    @pl.when(i == GRID - 1)
    def _():
        mean = sums_acc[...] / jnp.maximum(cnt_acc[...], 1.0)
        z = _lrelu(jnp.dot(mean, wr1_ref[...],
                           preferred_element_type=jnp.float32, precision=_HIGH))
        z = jnp.dot(z, wr2_ref[...],
                    preferred_element_type=jnp.float32, precision=_HIGH)
        out_ref[...] = 1.0 / (1.0 + jnp.exp(-z))


def _tc3_call(y0, y1, xpin, deg0, deg1, batch3d, W_r1, W_r2):
    return pl.pallas_call(
        _tc3_body,
        grid=(GRID,),
        in_specs=[
            pl.BlockSpec((BM, H), lambda i: (i, 0)),
            pl.BlockSpec((BM, H), lambda i: (i, 0)),
            pl.BlockSpec((BM, H), lambda i: (i, 0)),
            pl.BlockSpec((BM, DEGW), lambda i: (i, 0)),
            pl.BlockSpec((BM, DEGW), lambda i: (i, 0)),
            pl.BlockSpec((1, 1, BM), lambda i: (i, 0, 0)),
            pl.BlockSpec((H, H), lambda i: (0, 0)),
            pl.BlockSpec((H, O), lambda i: (0, 0)),
        ],
        out_specs=pl.BlockSpec((G, O), lambda i: (0, 0)),
        out_shape=jax.ShapeDtypeStruct((G, O), jnp.float32),
        scratch_shapes=[
            pltpu.VMEM((G, H), jnp.float32),
            pltpu.VMEM((G, H), jnp.float32),
        ],
        compiler_params=pltpu.CompilerParams(
            dimension_semantics=("arbitrary",)),
    )(y0, y1, xpin, deg0, deg1, batch3d, W_r1, W_r2)


def kernel(features, edge_index, batch, W_emb, W_m1, W_m2, W_r1, W_r2):
    row = edge_index[0]
    col = edge_index[1]
    n_extra = E_PAD - row.shape[0]
    pad_idx = jnp.arange(n_extra, dtype=jnp.int32)
    row_p = jnp.concatenate([row, pad_idx % N])
    col_p = jnp.concatenate([col, N + pad_idx % (N_PAD - N)])
    rows_l = row_p.reshape(32, CPT, CHUNK)
    cols_l = col_p.reshape(32, CPT, CHUNK)
    batch_p = jnp.concatenate(
        [batch, jnp.full((N_PAD - N,), G, jnp.int32)])
    batch3d = batch_p.reshape(GRID, 1, BM)

    ones_deg = jnp.ones((CHUNK, DEGW), jnp.float32)
    deg0, deg1 = _deg_kernel(cols_l, ones_deg)

    xp = _tc1_call(features, deg0, deg1, W_emb, W_m1)
    y0, y1 = _msg_kernel(xp, rows_l, cols_l)
    xp2 = _tc2_call(y0, y1, xp, deg0, deg1, W_m2)
    y0, y1 = _msg_kernel(xp2, rows_l, cols_l)
    return _tc3_call(y0, y1, xp2, deg0, deg1, batch3d, W_r1, W_r2)
